# baseline (speedup 1.0000x reference)
import functools

import jax
import jax.numpy as jnp
from jax import lax
from jax.experimental import pallas as pl
from jax.experimental.pallas import tpu as pltpu
from jax.experimental.pallas import tpu_sc as plsc

BATCH = 16384
DIM = 32
NUM_CORES = 2
NUM_SUBCORES = 16
NUM_WORKERS = NUM_CORES * NUM_SUBCORES
B_PER_W = BATCH // NUM_WORKERS
CHUNK = 128
NCHUNK = B_PER_W // CHUNK
LANES = 16
NGROUP = CHUNK // LANES
NACC = 4


def _mf_kernel(uidx_hbm, iidx_hbm, utab_hbm, itab_hbm, out_hbm,
               uidx_v, iidx_v, ubuf, ibuf, out_v, sems):
    wid = lax.axis_index("s") * NUM_CORES + lax.axis_index("c")
    base = wid * B_PER_W

    pltpu.sync_copy(uidx_hbm.at[pl.ds(base, B_PER_W)], uidx_v)
    pltpu.sync_copy(iidx_hbm.at[pl.ds(base, B_PER_W)], iidx_v)

    lane_iota = lax.iota(jnp.int32, LANES)

    def fire_chunk(c, slot):
        sem = sems.at[slot]

        def fire(g, carry):
            s = pl.ds(c * CHUNK + g * LANES, LANES)
            uvec = uidx_v[s]
            ivec = iidx_v[s]
            for j in range(LANES):
                dst = pl.ds(g * LANES + j, 1)
                pltpu.async_copy(utab_hbm.at[pl.ds(uvec[j], 1)],
                                 ubuf.at[slot].at[dst], sem)
                pltpu.async_copy(itab_hbm.at[pl.ds(ivec[j], 1)],
                                 ibuf.at[slot].at[dst], sem)
            return carry

        lax.fori_loop(0, NGROUP, fire, 0)

    def drain_chunk(slot):
        sem = sems.at[slot]
        pltpu.make_async_copy(utab_hbm.at[pl.ds(0, CHUNK)],
                              ubuf.at[slot], sem).wait()
        pltpu.make_async_copy(itab_hbm.at[pl.ds(0, CHUNK)],
                              ibuf.at[slot], sem).wait()

    def compute_chunk(c, slot):
        ub = ubuf.at[slot]
        ib = ibuf.at[slot]

        for g in range(NGROUP):
            rows = g * LANES + lane_iota
            accs = [jnp.zeros((LANES,), jnp.float32) for _ in range(NACC)]
            for d in range(DIM):
                cols = jnp.full((LANES,), d, jnp.int32)
                u = plsc.load_gather(ub, [rows, cols])
                v = plsc.load_gather(ib, [rows, cols])
                accs[d % NACC] = accs[d % NACC] + u * v
            acc = (accs[0] + accs[1]) + (accs[2] + accs[3])
            out_v[pl.ds(c * CHUNK + g * LANES, LANES)] = acc

    fire_chunk(0, 0)
    for c in range(NCHUNK):
        slot = c % 2
        if c + 1 < NCHUNK:
            fire_chunk(c + 1, 1 - slot)
        drain_chunk(slot)
        compute_chunk(c, slot)

    pltpu.sync_copy(out_v, out_hbm.at[pl.ds(base, B_PER_W)])


@jax.jit
def _mf(user_indices, item_indices, user_table, item_table):
    mesh = plsc.VectorSubcoreMesh(core_axis_name="c", subcore_axis_name="s")
    call = functools.partial(
        pl.kernel,
        mesh=mesh,
        out_type=jax.ShapeDtypeStruct((BATCH,), jnp.float32),
        scratch_types=[
            pltpu.VMEM((B_PER_W,), jnp.int32),
            pltpu.VMEM((B_PER_W,), jnp.int32),
            pltpu.VMEM((2, CHUNK, DIM), jnp.float32),
            pltpu.VMEM((2, CHUNK, DIM), jnp.float32),
            pltpu.VMEM((B_PER_W,), jnp.float32),
            pltpu.SemaphoreType.DMA((2,)),
        ],
        compiler_params=pltpu.CompilerParams(
            needs_layout_passes=False, use_tc_tiling_on_sc=True),
    )(_mf_kernel)
    return call(user_indices, item_indices, user_table, item_table)


def kernel(user_indices, item_indices, user_table, item_table):
    return _mf(user_indices.astype(jnp.int32), item_indices.astype(jnp.int32),
               user_table, item_table)


# speedup vs baseline: 1.0035x; 1.0035x over previous
import functools

import jax
import jax.numpy as jnp
from jax import lax
from jax.experimental import pallas as pl
from jax.experimental.pallas import tpu as pltpu
from jax.experimental.pallas import tpu_sc as plsc

BATCH = 16384
DIM = 32
NUM_CORES = 2
NUM_SUBCORES = 16
NUM_WORKERS = NUM_CORES * NUM_SUBCORES
B_PER_W = BATCH // NUM_WORKERS
CHUNK = 128
NCHUNK = B_PER_W // CHUNK
LANES = 16
NGROUP = CHUNK // LANES
NACC = 4


def _mf_kernel(uidx_hbm, iidx_hbm, utab_hbm, itab_hbm, out_hbm,
               uidx_v, iidx_v, ubuf, ibuf, out_v, sems):
    wid = lax.axis_index("s") * NUM_CORES + lax.axis_index("c")
    base = wid * B_PER_W

    pltpu.sync_copy(uidx_hbm.at[pl.ds(base, B_PER_W)], uidx_v)
    pltpu.sync_copy(iidx_hbm.at[pl.ds(base, B_PER_W)], iidx_v)

    lane_iota = lax.iota(jnp.int32, LANES)

    def fire_chunk(c, slot):
        sem = sems.at[slot]

        def fire(g, carry):
            s = pl.ds(c * CHUNK + g * LANES, LANES)
            uvec = uidx_v[s]
            ivec = iidx_v[s]
            for j in range(LANES):
                dst = pl.ds(g * LANES + j, 1)
                pltpu.async_copy(utab_hbm.at[pl.ds(uvec[j], 1)],
                                 ubuf.at[slot].at[dst], sem)
                pltpu.async_copy(itab_hbm.at[pl.ds(ivec[j], 1)],
                                 ibuf.at[slot].at[dst], sem)
            return carry

        lax.fori_loop(0, NGROUP, fire, 0)

    def drain_chunk(slot):
        sem = sems.at[slot]
        pltpu.make_async_copy(utab_hbm.at[pl.ds(0, CHUNK)],
                              ubuf.at[slot], sem).wait()
        pltpu.make_async_copy(itab_hbm.at[pl.ds(0, CHUNK)],
                              ibuf.at[slot], sem).wait()

    def compute_chunk(c, slot):
        ub = ubuf.at[slot]
        ib = ibuf.at[slot]

        def group_body(g, carry):
            rows = g * LANES + lane_iota
            accs = [jnp.zeros((LANES,), jnp.float32) for _ in range(NACC)]
            for d in range(DIM):
                cols = jnp.full((LANES,), d, jnp.int32)
                u = plsc.load_gather(ub, [rows, cols])
                v = plsc.load_gather(ib, [rows, cols])
                accs[d % NACC] = accs[d % NACC] + u * v
            acc = (accs[0] + accs[1]) + (accs[2] + accs[3])
            out_v[pl.ds(c * CHUNK + g * LANES, LANES)] = acc
            return carry

        lax.fori_loop(0, NGROUP, group_body, 0)

    fire_chunk(0, 0)
    for c in range(NCHUNK):
        slot = c % 2
        if c + 1 < NCHUNK:
            fire_chunk(c + 1, 1 - slot)
        drain_chunk(slot)
        compute_chunk(c, slot)

    pltpu.sync_copy(out_v, out_hbm.at[pl.ds(base, B_PER_W)])


@jax.jit
def _mf(user_indices, item_indices, user_table, item_table):
    mesh = plsc.VectorSubcoreMesh(core_axis_name="c", subcore_axis_name="s")
    call = functools.partial(
        pl.kernel,
        mesh=mesh,
        out_type=jax.ShapeDtypeStruct((BATCH,), jnp.float32),
        scratch_types=[
            pltpu.VMEM((B_PER_W,), jnp.int32),
            pltpu.VMEM((B_PER_W,), jnp.int32),
            pltpu.VMEM((2, CHUNK, DIM), jnp.float32),
            pltpu.VMEM((2, CHUNK, DIM), jnp.float32),
            pltpu.VMEM((B_PER_W,), jnp.float32),
            pltpu.SemaphoreType.DMA((2,)),
        ],
        compiler_params=pltpu.CompilerParams(
            needs_layout_passes=False, use_tc_tiling_on_sc=True),
    )(_mf_kernel)
    return call(user_indices, item_indices, user_table, item_table)


def kernel(user_indices, item_indices, user_table, item_table):
    return _mf(user_indices.astype(jnp.int32), item_indices.astype(jnp.int32),
               user_table, item_table)


# speedup vs baseline: 1.0234x; 1.0197x over previous
import functools

import jax
import jax.numpy as jnp
from jax import lax
from jax.experimental import pallas as pl
from jax.experimental.pallas import tpu as pltpu
from jax.experimental.pallas import tpu_sc as plsc

BATCH = 16384
DIM = 32
NUM_CORES = 2
NUM_SUBCORES = 16
NUM_WORKERS = NUM_CORES * NUM_SUBCORES
B_PER_W = BATCH // NUM_WORKERS
CHUNK = 128
NCHUNK = B_PER_W // CHUNK
LANES = 16
NGROUP = CHUNK // LANES
NACC = 4


def _mf_kernel(uidx_hbm, iidx_hbm, utab_hbm, itab_hbm, out_hbm,
               uidx_v, iidx_v, ubuf, ibuf, out_v, sems):
    wid = lax.axis_index("s") * NUM_CORES + lax.axis_index("c")
    base = wid * B_PER_W

    pltpu.sync_copy(uidx_hbm.at[pl.ds(base, B_PER_W)], uidx_v)
    pltpu.sync_copy(iidx_hbm.at[pl.ds(base, B_PER_W)], iidx_v)

    lane_iota = lax.iota(jnp.int32, LANES)

    def fire_chunk(c, slot):
        sem = sems.at[slot]

        def fire(g, carry):
            s = pl.ds(c * CHUNK + g * LANES, LANES)
            uvec = uidx_v[s]
            ivec = iidx_v[s]
            for j in range(LANES):
                dst = pl.ds(g * LANES + j, 1)
                pltpu.async_copy(utab_hbm.at[pl.ds(uvec[j], 1)],
                                 ubuf.at[slot].at[dst], sem)
                pltpu.async_copy(itab_hbm.at[pl.ds(ivec[j], 1)],
                                 ibuf.at[slot].at[dst], sem)
            return carry

        lax.fori_loop(0, NGROUP, fire, 0)

    def drain_chunk(slot):
        sem = sems.at[slot]
        pltpu.make_async_copy(utab_hbm.at[pl.ds(0, CHUNK)],
                              ubuf.at[slot], sem).wait()
        pltpu.make_async_copy(itab_hbm.at[pl.ds(0, CHUNK)],
                              ibuf.at[slot], sem).wait()

    def compute_chunk(c, slot):
        ub = ubuf.at[slot]
        ib = ibuf.at[slot]

        def group_body(g, carry):
            rows = g * LANES + lane_iota
            accs = [jnp.zeros((LANES,), jnp.float32) for _ in range(NACC)]
            for d in range(DIM):
                cols = (lane_iota + d) & (DIM - 1)
                u = plsc.load_gather(ub, [rows, cols])
                v = plsc.load_gather(ib, [rows, cols])
                accs[d % NACC] = accs[d % NACC] + u * v
            acc = (accs[0] + accs[1]) + (accs[2] + accs[3])
            out_v[pl.ds(c * CHUNK + g * LANES, LANES)] = acc
            return carry

        lax.fori_loop(0, NGROUP, group_body, 0)

    fire_chunk(0, 0)
    for c in range(NCHUNK):
        slot = c % 2
        if c + 1 < NCHUNK:
            fire_chunk(c + 1, 1 - slot)
        drain_chunk(slot)
        compute_chunk(c, slot)

    pltpu.sync_copy(out_v, out_hbm.at[pl.ds(base, B_PER_W)])


@jax.jit
def _mf(user_indices, item_indices, user_table, item_table):
    mesh = plsc.VectorSubcoreMesh(core_axis_name="c", subcore_axis_name="s")
    call = functools.partial(
        pl.kernel,
        mesh=mesh,
        out_type=jax.ShapeDtypeStruct((BATCH,), jnp.float32),
        scratch_types=[
            pltpu.VMEM((B_PER_W,), jnp.int32),
            pltpu.VMEM((B_PER_W,), jnp.int32),
            pltpu.VMEM((2, CHUNK, DIM), jnp.float32),
            pltpu.VMEM((2, CHUNK, DIM), jnp.float32),
            pltpu.VMEM((B_PER_W,), jnp.float32),
            pltpu.SemaphoreType.DMA((2,)),
        ],
        compiler_params=pltpu.CompilerParams(
            needs_layout_passes=False, use_tc_tiling_on_sc=True),
    )(_mf_kernel)
    return call(user_indices, item_indices, user_table, item_table)


def kernel(user_indices, item_indices, user_table, item_table):
    return _mf(user_indices.astype(jnp.int32), item_indices.astype(jnp.int32),
               user_table, item_table)
